# initial kernel scaffold (unmeasured)
import jax
import jax.numpy as jnp
from jax import lax
from jax.experimental import pallas as pl
from jax.experimental.pallas import tpu as pltpu


def kernel(O, Wo):
    B, S, H, D = O.shape
    HD = H * D
    N = Wo.shape[1]
    S_half = S // 2

    O2 = O.reshape(B, S, HD)

    def body(o_ref, wo_ref, out_ref, send_buf, recv_buf, send_sems, recv_sems):
        b = pl.program_id(0)
        my_x = lax.axis_index("x")
        my_y = lax.axis_index("y")
        my_z = lax.axis_index("z")
        peer_y = 1 - my_y
        peer = (my_x, peer_y, my_z)

        @pl.when(b == 0)
        def _():
            bar = pltpu.get_barrier_semaphore()
            pl.semaphore_signal(
                bar, inc=1, device_id=peer,
                device_id_type=pl.DeviceIdType.MESH,
            )
            pl.semaphore_wait(bar, 1)

        wo = wo_ref[...]

        x_peer = o_ref[0, pl.ds(peer_y * S_half, S_half), :]
        send_buf[...] = jnp.dot(x_peer, wo, preferred_element_type=jnp.float32)

        rdma = pltpu.make_async_remote_copy(
            src_ref=send_buf,
            dst_ref=recv_buf.at[b],
            send_sem=send_sems.at[b],
            recv_sem=recv_sems.at[b],
            device_id=peer,
            device_id_type=pl.DeviceIdType.MESH,
        )
        rdma.start()

        x_mine = o_ref[0, pl.ds(my_y * S_half, S_half), :]
        mine = jnp.dot(x_mine, wo, preferred_element_type=jnp.float32)

        rdma.wait()
        out_ref[0] = mine + recv_buf[b]

    return pl.pallas_call(
        body,
        grid=(B,),
        out_shape=jax.ShapeDtypeStruct((B, S_half, N), jnp.float32),
        in_specs=[
            pl.BlockSpec((1, S, HD), lambda b: (b, 0, 0)),
            pl.BlockSpec(memory_space=pltpu.VMEM),
        ],
        out_specs=pl.BlockSpec((1, S_half, N), lambda b: (b, 0, 0)),
        scratch_shapes=[
            pltpu.VMEM((S_half, N), jnp.float32),
            pltpu.VMEM((B, S_half, N), jnp.float32),
            pltpu.SemaphoreType.DMA((B,)),
            pltpu.SemaphoreType.DMA((B,)),
        ],
        compiler_params=pltpu.CompilerParams(
            collective_id=0,
            dimension_semantics=("arbitrary",),
        ),
    )(O2, Wo)


# baseline (device time: 539948 ns/iter reference)
import jax
import jax.numpy as jnp
from jax import lax
from jax.experimental import pallas as pl
from jax.experimental.pallas import tpu as pltpu

NC = 4


def kernel(O, Wo):
    B, S, H, D = O.shape
    HD = H * D
    N = Wo.shape[1]
    S_half = S // 2
    N_c = N // NC
    T = B * NC

    O2 = O.reshape(B, S, HD)

    def body(o_ref, wo_ref, out_ref, send_buf, recv_buf,
             send_sems, recv_sems, credit_sem):
        b = pl.program_id(0)
        n = pl.program_id(1)
        t = b * NC + n
        slot = t % 2
        my_x = lax.axis_index("x")
        my_y = lax.axis_index("y")
        my_z = lax.axis_index("z")
        peer_y = 1 - my_y
        peer = (my_x, peer_y, my_z)

        @pl.when(t == 0)
        def _():
            bar = pltpu.get_barrier_semaphore()
            pl.semaphore_signal(
                bar, inc=1, device_id=peer,
                device_id_type=pl.DeviceIdType.MESH,
            )
            pl.semaphore_wait(bar, 1)

        wo = wo_ref[...]

        x_peer = o_ref[0, pl.ds(peer_y * S_half, S_half), :]
        send_buf[...] = jnp.dot(x_peer, wo, preferred_element_type=jnp.float32)

        @pl.when(t >= 2)
        def _():
            pl.semaphore_wait(credit_sem, 1)

        rdma = pltpu.make_async_remote_copy(
            src_ref=send_buf,
            dst_ref=recv_buf.at[slot],
            send_sem=send_sems.at[slot],
            recv_sem=recv_sems.at[slot],
            device_id=peer,
            device_id_type=pl.DeviceIdType.MESH,
        )
        rdma.start()

        x_mine = o_ref[0, pl.ds(my_y * S_half, S_half), :]
        mine = jnp.dot(x_mine, wo, preferred_element_type=jnp.float32)

        rdma.wait()
        out_ref[0] = mine + recv_buf[slot]

        @pl.when(t < T - 2)
        def _():
            pl.semaphore_signal(
                credit_sem, inc=1, device_id=peer,
                device_id_type=pl.DeviceIdType.MESH,
            )

    return pl.pallas_call(
        body,
        grid=(B, NC),
        out_shape=jax.ShapeDtypeStruct((B, S_half, N), jnp.float32),
        in_specs=[
            pl.BlockSpec((1, S, HD), lambda b, n: (b, 0, 0)),
            pl.BlockSpec((HD, N_c), lambda b, n: (0, n)),
        ],
        out_specs=pl.BlockSpec((1, S_half, N_c), lambda b, n: (b, 0, n)),
        scratch_shapes=[
            pltpu.VMEM((S_half, N_c), jnp.float32),
            pltpu.VMEM((2, S_half, N_c), jnp.float32),
            pltpu.SemaphoreType.DMA((2,)),
            pltpu.SemaphoreType.DMA((2,)),
            pltpu.SemaphoreType.REGULAR,
        ],
        compiler_params=pltpu.CompilerParams(
            collective_id=0,
            dimension_semantics=("arbitrary", "arbitrary"),
            vmem_limit_bytes=58 * 1024 * 1024,
        ),
    )(O2, Wo)


# device time: 432020 ns/iter; 1.2498x vs baseline; 1.2498x over previous
import jax
import jax.numpy as jnp
from jax import lax
from jax.experimental import pallas as pl
from jax.experimental.pallas import tpu as pltpu

NC = 4


def kernel(O, Wo):
    B, S, H, D = O.shape
    HD = H * D
    N = Wo.shape[1]
    S_half = S // 2
    N_c = N // NC
    T = B * NC

    O2 = O.reshape(B, S, HD)

    def body(o_ref, wo_ref, out_ref, send_buf, recv_buf, res_buf,
             send_sems, recv_sems, credit_sem):
        t = pl.program_id(0)
        s_send = t % 2
        my_x = lax.axis_index("x")
        my_y = lax.axis_index("y")
        my_z = lax.axis_index("z")
        peer_y = 1 - my_y
        peer = (my_x, peer_y, my_z)

        def send_desc(slot):
            return pltpu.make_async_remote_copy(
                src_ref=send_buf.at[slot],
                dst_ref=recv_buf.at[slot % 3],
                send_sem=send_sems.at[slot],
                recv_sem=recv_sems.at[slot % 3],
                device_id=peer,
                device_id_type=pl.DeviceIdType.MESH,
            )

        def xfer_desc(chunk):
            return pltpu.make_async_remote_copy(
                src_ref=send_buf.at[chunk % 2],
                dst_ref=recv_buf.at[chunk % 3],
                send_sem=send_sems.at[chunk % 2],
                recv_sem=recv_sems.at[chunk % 3],
                device_id=peer,
                device_id_type=pl.DeviceIdType.MESH,
            )

        @pl.when(t == 0)
        def _():
            bar = pltpu.get_barrier_semaphore()
            pl.semaphore_signal(
                bar, inc=1, device_id=peer,
                device_id_type=pl.DeviceIdType.MESH,
            )
            pl.semaphore_wait(bar, 1)

        @pl.when(t < T)
        def _():
            @pl.when(t >= 2)
            def _():
                send_desc(s_send).wait_send()

            wo = wo_ref[...]

            x_peer = o_ref[0, pl.ds(peer_y * S_half, S_half), :]
            send_buf[s_send] = jnp.dot(x_peer, wo,
                                       preferred_element_type=jnp.float32)

            @pl.when(t >= 3)
            def _():
                pl.semaphore_wait(credit_sem, 1)

            xfer_desc(t).start()

            x_mine = o_ref[0, pl.ds(my_y * S_half, S_half), :]
            res_buf[s_send] = jnp.dot(x_mine, wo,
                                      preferred_element_type=jnp.float32)

        @pl.when(t >= 1)
        def _():
            c = t - 1
            xfer_desc(c).wait_recv()
            out_ref[0] = res_buf[c % 2] + recv_buf[c % 3]

            @pl.when(c < T - 3)
            def _():
                pl.semaphore_signal(
                    credit_sem, inc=1, device_id=peer,
                    device_id_type=pl.DeviceIdType.MESH,
                )

        @pl.when(t == T)
        def _():
            send_desc((T - 2) % 2).wait_send()
            send_desc((T - 1) % 2).wait_send()

    def o_map(t):
        c = jnp.minimum(t, T - 1)
        return (c // NC, 0, 0)

    def wo_map(t):
        c = jnp.minimum(t, T - 1)
        return (0, c % NC)

    def out_map(t):
        c = jnp.maximum(t - 1, 0)
        return (c // NC, 0, c % NC)

    return pl.pallas_call(
        body,
        grid=(T + 1,),
        out_shape=jax.ShapeDtypeStruct((B, S_half, N), jnp.float32),
        in_specs=[
            pl.BlockSpec((1, S, HD), o_map),
            pl.BlockSpec((HD, N_c), wo_map),
        ],
        out_specs=pl.BlockSpec((1, S_half, N_c), out_map),
        scratch_shapes=[
            pltpu.VMEM((2, S_half, N_c), jnp.float32),
            pltpu.VMEM((3, S_half, N_c), jnp.float32),
            pltpu.VMEM((2, S_half, N_c), jnp.float32),
            pltpu.SemaphoreType.DMA((2,)),
            pltpu.SemaphoreType.DMA((3,)),
            pltpu.SemaphoreType.REGULAR,
        ],
        compiler_params=pltpu.CompilerParams(
            collective_id=0,
            dimension_semantics=("arbitrary",),
            vmem_limit_bytes=58 * 1024 * 1024,
        ),
    )(O2, Wo)


# device time: 427989 ns/iter; 1.2616x vs baseline; 1.0094x over previous
import jax
import jax.numpy as jnp
from jax import lax
from jax.experimental import pallas as pl
from jax.experimental.pallas import tpu as pltpu

NC = 8


def kernel(O, Wo):
    B, S, H, D = O.shape
    HD = H * D
    N = Wo.shape[1]
    S_half = S // 2
    N_c = N // NC
    T = B * NC

    O2 = O.reshape(B, S, HD)

    def body(o_ref, wo_ref, out_ref, send_buf, recv_buf, res_buf,
             send_sems, recv_sems, credit_sem):
        t = pl.program_id(0)
        my_x = lax.axis_index("x")
        my_y = lax.axis_index("y")
        my_z = lax.axis_index("z")
        peer_y = 1 - my_y
        peer = (my_x, peer_y, my_z)

        def xfer_desc(chunk):
            return pltpu.make_async_remote_copy(
                src_ref=send_buf.at[chunk % 3],
                dst_ref=recv_buf.at[chunk % 4],
                send_sem=send_sems.at[chunk % 3],
                recv_sem=recv_sems.at[chunk % 4],
                device_id=peer,
                device_id_type=pl.DeviceIdType.MESH,
            )

        @pl.when(t == 0)
        def _():
            bar = pltpu.get_barrier_semaphore()
            pl.semaphore_signal(
                bar, inc=1, device_id=peer,
                device_id_type=pl.DeviceIdType.MESH,
            )
            pl.semaphore_wait(bar, 1)

        @pl.when(t < T)
        def _():
            @pl.when(t >= 3)
            def _():
                xfer_desc(t - 3).wait_send()

            wo = wo_ref[...]

            x_peer = o_ref[0, pl.ds(peer_y * S_half, S_half), :]
            send_buf[t % 3] = jnp.dot(x_peer, wo,
                                      preferred_element_type=jnp.float32)

            @pl.when(t >= 4)
            def _():
                pl.semaphore_wait(credit_sem, 1)

            xfer_desc(t).start()

            x_mine = o_ref[0, pl.ds(my_y * S_half, S_half), :]
            res_buf[t % 3] = jnp.dot(x_mine, wo,
                                     preferred_element_type=jnp.float32)

        @pl.when(t >= 2)
        def _():
            c = t - 2
            xfer_desc(c).wait_recv()
            out_ref[0] = res_buf[c % 3] + recv_buf[c % 4]

            @pl.when(c < T - 4)
            def _():
                pl.semaphore_signal(
                    credit_sem, inc=1, device_id=peer,
                    device_id_type=pl.DeviceIdType.MESH,
                )

        @pl.when(t == T + 1)
        def _():
            xfer_desc(T - 3).wait_send()
            xfer_desc(T - 2).wait_send()
            xfer_desc(T - 1).wait_send()

    def o_map(t):
        c = jnp.minimum(t, T - 1)
        return (c // NC, 0, 0)

    def wo_map(t):
        c = jnp.minimum(t, T - 1)
        return (0, c % NC)

    def out_map(t):
        c = jnp.clip(t - 2, 0, T - 1)
        return (c // NC, 0, c % NC)

    return pl.pallas_call(
        body,
        grid=(T + 2,),
        out_shape=jax.ShapeDtypeStruct((B, S_half, N), jnp.float32),
        in_specs=[
            pl.BlockSpec((1, S, HD), o_map),
            pl.BlockSpec((HD, N_c), wo_map),
        ],
        out_specs=pl.BlockSpec((1, S_half, N_c), out_map),
        scratch_shapes=[
            pltpu.VMEM((3, S_half, N_c), jnp.float32),
            pltpu.VMEM((4, S_half, N_c), jnp.float32),
            pltpu.VMEM((3, S_half, N_c), jnp.float32),
            pltpu.SemaphoreType.DMA((3,)),
            pltpu.SemaphoreType.DMA((4,)),
            pltpu.SemaphoreType.REGULAR,
        ],
        compiler_params=pltpu.CompilerParams(
            collective_id=0,
            dimension_semantics=("arbitrary",),
            vmem_limit_bytes=58 * 1024 * 1024,
        ),
    )(O2, Wo)
